# baseline (speedup 1.0000x reference)
import functools

import jax
import jax.numpy as jnp
from jax import lax
from jax.experimental import pallas as pl
from jax.experimental.pallas import tpu as pltpu
from jax.experimental.pallas import tpu_sc as plsc


def _router_body(nexp, x_ref, wr_ref, idx_ref, w_ref):
    rb = x_ref.shape[0]
    xb = x_ref[...].astype(jnp.bfloat16)
    wb = wr_ref[...].astype(jnp.bfloat16)
    logits = lax.dot_general(xb, wb, (((1,), (1,)), ((), ())),
                             preferred_element_type=jnp.float32)
    lane = lax.broadcasted_iota(jnp.int32, (rb, 128), 1)
    valid = lane < nexp
    neg = jnp.float32(-1e30)
    lm = jnp.where(valid, logits, neg)
    m1 = jnp.max(lm, axis=1, keepdims=True)
    i1 = jnp.min(jnp.where(lm >= m1, lane, 128), axis=1, keepdims=True)
    lm2 = jnp.where(lane == i1, neg, lm)
    m2 = jnp.max(lm2, axis=1, keepdims=True)
    i2 = jnp.min(jnp.where(lm2 >= m2, lane, 128), axis=1, keepdims=True)
    w1v = jax.nn.sigmoid(m1 - m2)
    idx_ref[...] = jnp.where(lane == 0, i1, jnp.where(lane == 1, i2, 0))
    w_ref[...] = jnp.where(lane == 0, w1v,
                           jnp.where(lane == 1, 1.0 - w1v, 0.0))


def _mm_body(nf, blk, nblk_ref, bofs_ref, x_ref, w1_ref, v1_ref, w2_ref,
             out_ref, w1b_ref, v1b_ref, w2b_ref):
    j = pl.program_id(0)
    e = pl.program_id(1)

    w1b_ref[...] = w1_ref[0].astype(jnp.bfloat16)
    v1b_ref[...] = v1_ref[0].astype(jnp.bfloat16)
    w2b_ref[...] = w2_ref[0].astype(jnp.bfloat16)

    base = bofs_ref[e]

    fc = w1b_ref.shape[0]
    strip = 512

    def one_block(r0):
        x = x_ref[pl.ds(r0, blk), :]
        for s in range(fc // strip):
            cs = pl.ds(s * strip, strip)
            h1 = lax.dot_general(x, w1b_ref[cs, :], (((1,), (1,)), ((), ())),
                                 preferred_element_type=jnp.float32)
            hv = lax.dot_general(x, v1b_ref[cs, :], (((1,), (1,)), ((), ())),
                                 preferred_element_type=jnp.float32)
            h = (h1 * jax.nn.sigmoid(h1)) * hv
            y = lax.dot_general(h.astype(jnp.bfloat16), w2b_ref[cs, :],
                                (((1,), (0,)), ((), ())),
                                preferred_element_type=jnp.float32)
            if s == 0:
                @pl.when(j == 0)
                def _():
                    out_ref[pl.ds(r0, blk), :] = y

                @pl.when(j > 0)
                def _():
                    out_ref[pl.ds(r0, blk), :] += y
            else:
                out_ref[pl.ds(r0, blk), :] += y

    def block_step(t, carry):
        one_block(pl.multiple_of((base + t) * blk, blk))
        return carry

    lax.fori_loop(0, nblk_ref[e], block_step, 0)


def kernel(hidden_states, Wr, W1, V1, W2):
    B, S, H = hidden_states.shape
    E, F, _ = W1.shape
    T = B * S
    K = 2

    xf = jnp.swapaxes(hidden_states, 0, 1).reshape(T, H)

    RB = 256
    Wrp = jnp.zeros((128, H), jnp.float32).at[:E].set(Wr)
    eiw, wts = pl.pallas_call(
        functools.partial(_router_body, E),
        grid=(T // RB,),
        in_specs=[pl.BlockSpec((RB, H), lambda i: (i, 0)),
                  pl.BlockSpec((128, H), lambda i: (0, 0))],
        out_specs=[pl.BlockSpec((RB, 128), lambda i: (i, 0)),
                   pl.BlockSpec((RB, 128), lambda i: (i, 0))],
        out_shape=[jax.ShapeDtypeStruct((T, 128), jnp.int32),
                   jax.ShapeDtypeStruct((T, 128), jnp.float32)],
    )(xf, Wrp)
    e1, e2 = eiw[:, 0], eiw[:, 1]
    w1, w2 = wts[:, 0], wts[:, 1]

    BLK = 128
    NB = -(-(T * K + E * (BLK - 1)) // BLK)
    P = NB * BLK
    ar = jnp.arange(E)
    oh1 = (e1[:, None] == ar).astype(jnp.int32)
    oh2 = (e2[:, None] == ar).astype(jnp.int32)
    c1 = jnp.cumsum(oh1, axis=0)
    c2 = jnp.cumsum(oh2, axis=0)
    n1 = c1[-1]
    cnt = n1 + c2[-1]
    nblk = ((cnt + BLK - 1) // BLK).astype(jnp.int32)
    cumblk = jnp.cumsum(nblk)
    bofs = (cumblk - nblk).astype(jnp.int32)
    goff = (cumblk - nblk) * BLK
    rank1 = jnp.sum(c1 * oh1, axis=1) - 1
    rank2 = jnp.sum((n1[None, :] + c2) * oh2, axis=1) - 1
    pos1 = (jnp.sum(goff[None, :] * oh1, axis=1) + rank1).astype(jnp.int32)
    pos2 = (jnp.sum(goff[None, :] * oh2, axis=1) + rank2).astype(jnp.int32)
    pk = jnp.stack([pos1, pos2,
                    lax.bitcast_convert_type(w1, jnp.int32),
                    lax.bitcast_convert_type(w2, jnp.int32)], axis=0)

    info = plsc.get_sparse_core_info()
    NW = info.num_cores * info.num_subcores
    ncores = info.num_cores
    mesh = plsc.VectorSubcoreMesh(core_axis_name="c", subcore_axis_name="s")
    Hw = H // 2
    xb32 = lax.bitcast_convert_type(
        xf.astype(jnp.bfloat16).reshape(T, Hw, 2), jnp.int32)
    tpw = T // NW
    nsc = -(-tpw // 128)
    sch = tpw // nsc

    def scatter_body(x_hbm, pk_hbm, out_hbm, i1_v, i2_v, rows_v,
                     sem1, sem2, sem3):
        wid = lax.axis_index("s") * ncores + lax.axis_index("c")
        for c in range(nsc):
            base = wid * tpw + c * sch
            ld1 = pltpu.async_copy(pk_hbm.at[0, pl.ds(base, sch)], i1_v, sem1)
            ld2 = pltpu.async_copy(pk_hbm.at[1, pl.ds(base, sch)], i2_v, sem2)
            ld3 = pltpu.async_copy(x_hbm.at[pl.ds(base, sch)], rows_v, sem3)
            ld1.wait()
            ld2.wait()
            ld3.wait()
            cp1 = pltpu.async_copy(rows_v, out_hbm.at[i1_v], sem1)
            cp2 = pltpu.async_copy(rows_v, out_hbm.at[i2_v], sem2)
            cp1.wait()
            cp2.wait()

    x_sorted32 = pl.kernel(
        scatter_body,
        out_type=jax.ShapeDtypeStruct((P, Hw), jnp.int32),
        mesh=mesh,
        scratch_types=[pltpu.VMEM((sch,), jnp.int32),
                       pltpu.VMEM((sch,), jnp.int32),
                       pltpu.VMEM((sch, Hw), jnp.int32),
                       pltpu.SemaphoreType.DMA,
                       pltpu.SemaphoreType.DMA,
                       pltpu.SemaphoreType.DMA],
    )(xb32, pk)
    x_sorted = lax.bitcast_convert_type(
        x_sorted32, jnp.bfloat16).reshape(P, H)

    FC = 1024
    NF = F // FC
    y_sorted = pl.pallas_call(
        functools.partial(_mm_body, NF, BLK),
        grid_spec=pltpu.PrefetchScalarGridSpec(
            num_scalar_prefetch=2,
            grid=(NF, E),
            in_specs=[
                pl.BlockSpec((P, H), lambda j, e, nb, bo: (0, 0)),
                pl.BlockSpec((1, FC, H), lambda j, e, nb, bo: (e, j, 0)),
                pl.BlockSpec((1, FC, H), lambda j, e, nb, bo: (e, j, 0)),
                pl.BlockSpec((1, FC, H), lambda j, e, nb, bo: (e, j, 0)),
            ],
            out_specs=pl.BlockSpec((P, H), lambda j, e, nb, bo: (0, 0)),
            scratch_shapes=[pltpu.VMEM((FC, H), jnp.bfloat16),
                            pltpu.VMEM((FC, H), jnp.bfloat16),
                            pltpu.VMEM((FC, H), jnp.bfloat16)],
        ),
        out_shape=jax.ShapeDtypeStruct((P, H), jnp.float32),
        compiler_params=pltpu.CompilerParams(
            dimension_semantics=("arbitrary", "arbitrary"),
            vmem_limit_bytes=120 * 1024 * 1024,
        ),
    )(nblk, bofs, x_sorted, W1, V1, W2)

    ncc = -(-tpw // 32)
    cch = tpw // ncc
    nq = H // 16

    def comb_body(y_hbm, pk_hbm, out_hbm,
                  i1_v, i2_v, w1_v, w2_v, r1_v, r2_v, sem1, sem2):
        wid = lax.axis_index("s") * ncores + lax.axis_index("c")
        for c in range(ncc):
            base = wid * tpw + c * cch
            ld1 = pltpu.async_copy(pk_hbm.at[0, pl.ds(base, cch)], i1_v, sem1)
            ld2 = pltpu.async_copy(pk_hbm.at[1, pl.ds(base, cch)], i2_v, sem2)
            ld3 = pltpu.async_copy(pk_hbm.at[2, pl.ds(base, cch)], w1_v, sem1)
            ld4 = pltpu.async_copy(pk_hbm.at[3, pl.ds(base, cch)], w2_v, sem2)
            ld1.wait()
            ld2.wait()
            ld3.wait()
            ld4.wait()
            cp1 = pltpu.async_copy(y_hbm.at[i1_v], r1_v, sem1)
            cp2 = pltpu.async_copy(y_hbm.at[i2_v], r2_v, sem2)
            cp1.wait()
            cp2.wait()

            def row_comb(r, carry):
                rv = jnp.full((16,), r, jnp.int32)
                wa = plsc.bitcast(plsc.load_gather(w1_v, [rv]), jnp.float32)
                wb = plsc.bitcast(plsc.load_gather(w2_v, [rv]), jnp.float32)
                for q in range(nq):
                    sl = pl.ds(q * 16, 16)
                    r1_v[r, sl] = r1_v[r, sl] * wa + r2_v[r, sl] * wb
                return carry

            lax.fori_loop(0, cch, row_comb, 0)
            pltpu.sync_copy(r1_v, out_hbm.at[pl.ds(base, cch)])

    out_flat = pl.kernel(
        comb_body,
        out_type=jax.ShapeDtypeStruct((T, H), jnp.float32),
        mesh=mesh,
        compiler_params=pltpu.CompilerParams(needs_layout_passes=False),
        scratch_types=[pltpu.VMEM((cch,), jnp.int32),
                       pltpu.VMEM((cch,), jnp.int32),
                       pltpu.VMEM((cch,), jnp.int32),
                       pltpu.VMEM((cch,), jnp.int32),
                       pltpu.VMEM((cch, H), jnp.float32),
                       pltpu.VMEM((cch, H), jnp.float32),
                       pltpu.SemaphoreType.DMA,
                       pltpu.SemaphoreType.DMA],
    )(y_sorted, pk)

    return jnp.swapaxes(out_flat.reshape(S, B, H), 0, 1)


# speedup vs baseline: 1.1030x; 1.1030x over previous
import functools

import jax
import jax.numpy as jnp
from jax import lax
from jax.experimental import pallas as pl
from jax.experimental.pallas import tpu as pltpu
from jax.experimental.pallas import tpu_sc as plsc


def _router_body(nexp, x_ref, wr_ref, idx_ref, w_ref):
    rb = x_ref.shape[0]
    xb = x_ref[...].astype(jnp.bfloat16)
    wb = wr_ref[...].astype(jnp.bfloat16)
    logits = lax.dot_general(xb, wb, (((1,), (1,)), ((), ())),
                             preferred_element_type=jnp.float32)
    lane = lax.broadcasted_iota(jnp.int32, (rb, 128), 1)
    valid = lane < nexp
    neg = jnp.float32(-1e30)
    lm = jnp.where(valid, logits, neg)
    m1 = jnp.max(lm, axis=1, keepdims=True)
    i1 = jnp.min(jnp.where(lm >= m1, lane, 128), axis=1, keepdims=True)
    lm2 = jnp.where(lane == i1, neg, lm)
    m2 = jnp.max(lm2, axis=1, keepdims=True)
    i2 = jnp.min(jnp.where(lm2 >= m2, lane, 128), axis=1, keepdims=True)
    w1v = jax.nn.sigmoid(m1 - m2)
    idx_ref[...] = jnp.where(lane == 0, i1, jnp.where(lane == 1, i2, 0))
    w_ref[...] = jnp.where(lane == 0, w1v,
                           jnp.where(lane == 1, 1.0 - w1v, 0.0))


def _mm_body(nf, blk, nblk_ref, bofs_ref, x_ref, w1_ref, v1_ref, w2_ref,
             out_ref, w1b_ref, v1b_ref, w2b_ref):
    j = pl.program_id(0)
    e = pl.program_id(1)

    w1b_ref[...] = w1_ref[0].astype(jnp.bfloat16)
    v1b_ref[...] = v1_ref[0].astype(jnp.bfloat16)
    w2b_ref[...] = w2_ref[0].astype(jnp.bfloat16)

    base = bofs_ref[e]

    def one_block(r0):
        x = x_ref[pl.ds(r0, blk), :]
        h1 = lax.dot_general(x, w1b_ref[...], (((1,), (1,)), ((), ())),
                             preferred_element_type=jnp.float32)
        hv = lax.dot_general(x, v1b_ref[...], (((1,), (1,)), ((), ())),
                             preferred_element_type=jnp.float32)
        h = (h1 * jax.nn.sigmoid(h1)) * hv
        y = lax.dot_general(h.astype(jnp.bfloat16), w2b_ref[...],
                            (((1,), (0,)), ((), ())),
                            preferred_element_type=jnp.float32)

        @pl.when(j == 0)
        def _():
            out_ref[pl.ds(r0, blk), :] = y

        @pl.when(j > 0)
        def _():
            out_ref[pl.ds(r0, blk), :] += y

    def block_step(t, carry):
        one_block(pl.multiple_of((base + t) * blk, blk))
        return carry

    lax.fori_loop(0, nblk_ref[e], block_step, 0)


def kernel(hidden_states, Wr, W1, V1, W2):
    B, S, H = hidden_states.shape
    E, F, _ = W1.shape
    T = B * S
    K = 2

    xf = jnp.swapaxes(hidden_states, 0, 1).reshape(T, H)

    RB = 256
    Wrp = jnp.zeros((128, H), jnp.float32).at[:E].set(Wr)
    eiw, wts = pl.pallas_call(
        functools.partial(_router_body, E),
        grid=(T // RB,),
        in_specs=[pl.BlockSpec((RB, H), lambda i: (i, 0)),
                  pl.BlockSpec((128, H), lambda i: (0, 0))],
        out_specs=[pl.BlockSpec((RB, 128), lambda i: (i, 0)),
                   pl.BlockSpec((RB, 128), lambda i: (i, 0))],
        out_shape=[jax.ShapeDtypeStruct((T, 128), jnp.int32),
                   jax.ShapeDtypeStruct((T, 128), jnp.float32)],
    )(xf, Wrp)
    e1, e2 = eiw[:, 0], eiw[:, 1]
    w1, w2 = wts[:, 0], wts[:, 1]

    BLK = 128
    NB = -(-(T * K + E * (BLK - 1)) // BLK)
    P = NB * BLK
    ar = jnp.arange(E)
    oh1 = (e1[:, None] == ar).astype(jnp.int32)
    oh2 = (e2[:, None] == ar).astype(jnp.int32)
    c1 = jnp.cumsum(oh1, axis=0)
    c2 = jnp.cumsum(oh2, axis=0)
    n1 = c1[-1]
    cnt = n1 + c2[-1]
    nblk = ((cnt + BLK - 1) // BLK).astype(jnp.int32)
    cumblk = jnp.cumsum(nblk)
    bofs = (cumblk - nblk).astype(jnp.int32)
    goff = (cumblk - nblk) * BLK
    rank1 = jnp.sum(c1 * oh1, axis=1) - 1
    rank2 = jnp.sum((n1[None, :] + c2) * oh2, axis=1) - 1
    pos1 = (jnp.sum(goff[None, :] * oh1, axis=1) + rank1).astype(jnp.int32)
    pos2 = (jnp.sum(goff[None, :] * oh2, axis=1) + rank2).astype(jnp.int32)
    pk = jnp.stack([pos1, pos2,
                    lax.bitcast_convert_type(w1, jnp.int32),
                    lax.bitcast_convert_type(w2, jnp.int32)], axis=0)

    info = plsc.get_sparse_core_info()
    NW = info.num_cores * info.num_subcores
    ncores = info.num_cores
    mesh = plsc.VectorSubcoreMesh(core_axis_name="c", subcore_axis_name="s")
    Hw = H // 2
    xb32 = lax.bitcast_convert_type(
        xf.astype(jnp.bfloat16).reshape(T, Hw, 2), jnp.int32)
    tpw = T // NW
    nsc = -(-tpw // 128)
    sch = tpw // nsc

    def scatter_body(x_hbm, pk_hbm, out_hbm, i1_v, i2_v, rows_v,
                     sem1, sem2, sem3):
        wid = lax.axis_index("s") * ncores + lax.axis_index("c")
        for c in range(nsc):
            base = wid * tpw + c * sch
            ld1 = pltpu.async_copy(pk_hbm.at[0, pl.ds(base, sch)], i1_v, sem1)
            ld2 = pltpu.async_copy(pk_hbm.at[1, pl.ds(base, sch)], i2_v, sem2)
            ld3 = pltpu.async_copy(x_hbm.at[pl.ds(base, sch)], rows_v, sem3)
            ld1.wait()
            ld2.wait()
            ld3.wait()
            cp1 = pltpu.async_copy(rows_v, out_hbm.at[i1_v], sem1)
            cp2 = pltpu.async_copy(rows_v, out_hbm.at[i2_v], sem2)
            cp1.wait()
            cp2.wait()

    x_sorted32 = pl.kernel(
        scatter_body,
        out_type=jax.ShapeDtypeStruct((P, Hw), jnp.int32),
        mesh=mesh,
        scratch_types=[pltpu.VMEM((sch,), jnp.int32),
                       pltpu.VMEM((sch,), jnp.int32),
                       pltpu.VMEM((sch, Hw), jnp.int32),
                       pltpu.SemaphoreType.DMA,
                       pltpu.SemaphoreType.DMA,
                       pltpu.SemaphoreType.DMA],
    )(xb32, pk)
    x_sorted = lax.bitcast_convert_type(
        x_sorted32, jnp.bfloat16).reshape(P, H)

    FC = 1024
    NF = F // FC
    y_sorted = pl.pallas_call(
        functools.partial(_mm_body, NF, BLK),
        grid_spec=pltpu.PrefetchScalarGridSpec(
            num_scalar_prefetch=2,
            grid=(NF, E),
            in_specs=[
                pl.BlockSpec((P, H), lambda j, e, nb, bo: (0, 0)),
                pl.BlockSpec((1, FC, H), lambda j, e, nb, bo: (e, j, 0)),
                pl.BlockSpec((1, FC, H), lambda j, e, nb, bo: (e, j, 0)),
                pl.BlockSpec((1, FC, H), lambda j, e, nb, bo: (e, j, 0)),
            ],
            out_specs=pl.BlockSpec((P, H), lambda j, e, nb, bo: (0, 0)),
            scratch_shapes=[pltpu.VMEM((FC, H), jnp.bfloat16),
                            pltpu.VMEM((FC, H), jnp.bfloat16),
                            pltpu.VMEM((FC, H), jnp.bfloat16)],
        ),
        out_shape=jax.ShapeDtypeStruct((P, H), jnp.float32),
        compiler_params=pltpu.CompilerParams(
            dimension_semantics=("arbitrary", "arbitrary"),
            vmem_limit_bytes=120 * 1024 * 1024,
        ),
    )(nblk, bofs, x_sorted, W1, V1, W2)

    ncc = -(-tpw // 32)
    cch = tpw // ncc
    nq = H // 16

    def comb_body(y_hbm, pk_hbm, out_hbm,
                  i1_v, i2_v, w1_v, w2_v, r1_v, r2_v, sem1, sem2):
        wid = lax.axis_index("s") * ncores + lax.axis_index("c")
        for c in range(ncc):
            base = wid * tpw + c * cch
            ld1 = pltpu.async_copy(pk_hbm.at[0, pl.ds(base, cch)], i1_v, sem1)
            ld2 = pltpu.async_copy(pk_hbm.at[1, pl.ds(base, cch)], i2_v, sem2)
            ld3 = pltpu.async_copy(pk_hbm.at[2, pl.ds(base, cch)], w1_v, sem1)
            ld4 = pltpu.async_copy(pk_hbm.at[3, pl.ds(base, cch)], w2_v, sem2)
            ld1.wait()
            ld2.wait()
            ld3.wait()
            ld4.wait()
            cp1 = pltpu.async_copy(y_hbm.at[i1_v], r1_v, sem1)
            cp2 = pltpu.async_copy(y_hbm.at[i2_v], r2_v, sem2)
            cp1.wait()
            cp2.wait()

            def row_comb(r, carry):
                rv = jnp.full((16,), r, jnp.int32)
                wa = plsc.bitcast(plsc.load_gather(w1_v, [rv]), jnp.float32)
                wb = plsc.bitcast(plsc.load_gather(w2_v, [rv]), jnp.float32)
                for q in range(nq):
                    sl = pl.ds(q * 16, 16)
                    r1_v[r, sl] = r1_v[r, sl] * wa + r2_v[r, sl] * wb
                return carry

            lax.fori_loop(0, cch, row_comb, 0)
            pltpu.sync_copy(r1_v, out_hbm.at[pl.ds(base, cch)])

    out_flat = pl.kernel(
        comb_body,
        out_type=jax.ShapeDtypeStruct((T, H), jnp.float32),
        mesh=mesh,
        compiler_params=pltpu.CompilerParams(needs_layout_passes=False),
        scratch_types=[pltpu.VMEM((cch,), jnp.int32),
                       pltpu.VMEM((cch,), jnp.int32),
                       pltpu.VMEM((cch,), jnp.int32),
                       pltpu.VMEM((cch,), jnp.int32),
                       pltpu.VMEM((cch, H), jnp.float32),
                       pltpu.VMEM((cch, H), jnp.float32),
                       pltpu.SemaphoreType.DMA,
                       pltpu.SemaphoreType.DMA],
    )(y_sorted, pk)

    return jnp.swapaxes(out_flat.reshape(S, B, H), 0, 1)
